# Initial kernel scaffold; baseline (speedup 1.0000x reference)
import jax
import jax.numpy as jnp
from jax.experimental import pallas as pl


def kernel(x, batch, W, b):
    raise NotImplementedError("write your pallas kernel here")


# speedup vs baseline: 1.5202x; 1.5202x over previous
import functools

import jax
import jax.numpy as jnp
from jax import lax
from jax.experimental import pallas as pl
from jax.experimental.pallas import tpu as pltpu
from jax.experimental.pallas import tpu_sc as plsc

N = 100000
D = 128
S = 64
BLK = 160
NBLK = N // BLK
NW = 32
GRP = BLK // 16
NEG = -1e30


def _wid():
    return lax.axis_index("s") * 2 + lax.axis_index("c")


def _nblk(wid):
    return (NBLK - wid + NW - 1) // NW


def _mz_combine(mzv):
    iota = lax.iota(jnp.int32, 16)
    zero = jnp.zeros((16,), jnp.int32)
    one = zero + 1
    m1 = plsc.load_gather(mzv, [iota, zero])
    m2 = plsc.load_gather(mzv, [iota + 16, zero])
    z1 = plsc.load_gather(mzv, [iota, one])
    z2 = plsc.load_gather(mzv, [iota + 16, one])
    m = jnp.maximum(jnp.max(m1), jnp.max(m2))
    mv = jnp.full((16,), m, jnp.float32)
    z = jnp.sum(z1 * jnp.exp(m1 - mv) + z2 * jnp.exp(m2 - mv))
    zv = jnp.full((16,), z, jnp.float32)
    invzv = jnp.ones((16,), jnp.float32) / zv
    return mv, invzv


def _k1_body(x_hbm, wb_hbm, w_hbm, mz_hbm, xv, wblkv, wbv, mzv):
    wid = _wid()
    pltpu.sync_copy(wb_hbm, wbv)
    iota = lax.iota(jnp.int32, 16)
    rowv = [iota + 16 * g for g in range(GRP)]
    m0 = jnp.full((16,), NEG, jnp.float32)
    z0 = jnp.zeros((16,), jnp.float32)

    def blk_body(i, carry):
        m, z = carry
        r0 = (wid + i * NW) * BLK
        pltpu.sync_copy(x_hbm.at[pl.ds(r0, BLK)], xv)

        def j_body(j, accs):
            colv = jnp.full((16,), j, jnp.int32)
            wj = plsc.load_gather(wbv, [colv])
            return tuple(
                acc + plsc.load_gather(xv, [rowv[g], colv]) * wj
                for g, acc in enumerate(accs)
            )

        accs = lax.fori_loop(
            0, D, j_body, tuple(jnp.zeros((16,), jnp.float32) for _ in range(GRP))
        )
        bias = plsc.load_gather(wbv, [jnp.full((16,), D, jnp.int32)])
        for g in range(GRP):
            wvec = accs[g] + bias
            wblkv[pl.ds(16 * g, 16)] = wvec
            mn = jnp.maximum(m, wvec)
            z = z * jnp.exp(m - mn) + jnp.exp(wvec - mn)
            m = mn
        pltpu.sync_copy(wblkv, w_hbm.at[pl.ds(r0, BLK)])
        return (m, z)

    m, z = lax.fori_loop(0, _nblk(wid), blk_body, (m0, z0))
    mw = jnp.max(m)
    zw = jnp.sum(z * jnp.exp(m - mw))
    mzv[...] = jnp.where(iota == 0, mw, jnp.where(iota == 1, zw, 0.0))
    pltpu.sync_copy(mzv, mz_hbm.at[wid])


def _k2_body(w_hbm, b_hbm, mz_hbm, dp_hbm, wv, bv, mzv, denomv):
    wid = _wid()
    pltpu.sync_copy(mz_hbm, mzv)
    mv, invzv = _mz_combine(mzv)
    zero16 = jnp.zeros((16,), jnp.float32)
    for c in range(S // 16):
        denomv[pl.ds(16 * c, 16)] = zero16

    def blk_body(i, _):
        r0 = (wid + i * NW) * BLK
        pltpu.sync_copy(w_hbm.at[pl.ds(r0, BLK)], wv)
        pltpu.sync_copy(b_hbm.at[pl.ds(r0, BLK)], bv)
        for g in range(GRP):
            wvec = wv[pl.ds(16 * g, 16)]
            ev = jnp.exp(jnp.exp(wvec - mv) * invzv)
            plsc.addupdate_scatter(denomv, [bv[pl.ds(16 * g, 16)]], ev)
        return 0

    lax.fori_loop(0, _nblk(wid), blk_body, 0)
    pltpu.sync_copy(denomv, dp_hbm.at[wid])


def _k3_body(x_hbm, w_hbm, b_hbm, mz_hbm, dp_hbm, acc_hbm,
             xv, wv, bv, mzv, dpv, cinvv, accv):
    wid = _wid()
    pltpu.sync_copy(mz_hbm, mzv)
    mv, invzv = _mz_combine(mzv)
    pltpu.sync_copy(dp_hbm, dpv)
    one16 = jnp.ones((16,), jnp.float32)
    for c in range(S // 16):
        s = dpv[0, pl.ds(16 * c, 16)]
        for r in range(1, NW):
            s = s + dpv[r, pl.ds(16 * c, 16)]
        cinvv[pl.ds(16 * c, 16)] = one16 / s

    zero16 = jnp.zeros((16,), jnp.float32)

    def zero_body(r, _):
        for j in range(D // 16):
            accv[r, pl.ds(16 * j, 16)] = zero16
        return 0

    lax.fori_loop(0, S, zero_body, 0)

    def blk_body(i, _):
        r0 = (wid + i * NW) * BLK
        pltpu.sync_copy(x_hbm.at[pl.ds(r0, BLK)], xv)
        pltpu.sync_copy(w_hbm.at[pl.ds(r0, BLK)], wv)
        pltpu.sync_copy(b_hbm.at[pl.ds(r0, BLK)], bv)

        def grp_body(g, _):
            wvec = wv[pl.ds(16 * g, 16)]
            bvec = bv[pl.ds(16 * g, 16)]
            gv = jnp.exp(wvec - mv) * invzv
            nw = jnp.exp(gv) * plsc.load_gather(cinvv, [bvec])
            for l in range(16):
                bi = bvec[l]
                sv = jnp.full((16,), nw[l], jnp.float32)
                r = 16 * g + l
                for j in range(D // 16):
                    plsc.addupdate(accv.at[bi, pl.ds(16 * j, 16)],
                                   xv[r, pl.ds(16 * j, 16)] * sv)
            return 0

        lax.fori_loop(0, GRP, grp_body, 0)
        return 0

    lax.fori_loop(0, _nblk(wid), blk_body, 0)
    pltpu.sync_copy(accv, acc_hbm.at[wid])


def _k4_body(a_ref, o_ref):
    o_ref[...] = jnp.sum(a_ref[...], axis=0)


@jax.jit
def kernel(x, batch, W, b):
    f32 = jnp.float32
    wb = jnp.concatenate(
        [W.reshape(D).astype(f32), jnp.broadcast_to(b.astype(f32), (32,))]
    )
    mesh = plsc.VectorSubcoreMesh(core_axis_name="c", subcore_axis_name="s")

    k1 = pl.kernel(
        _k1_body,
        out_type=(
            jax.ShapeDtypeStruct((N,), f32),
            jax.ShapeDtypeStruct((NW, 16), f32),
        ),
        mesh=mesh,
        compiler_params=pltpu.CompilerParams(needs_layout_passes=False),
        scratch_types=[
            pltpu.VMEM((BLK, D), f32),
            pltpu.VMEM((BLK,), f32),
            pltpu.VMEM((D + 32,), f32),
            pltpu.VMEM((16,), f32),
        ],
    )
    w_arr, mz = k1(x, wb)

    k2 = pl.kernel(
        _k2_body,
        out_type=jax.ShapeDtypeStruct((NW, S), f32),
        mesh=mesh,
        compiler_params=pltpu.CompilerParams(needs_layout_passes=False),
        scratch_types=[
            pltpu.VMEM((BLK,), f32),
            pltpu.VMEM((BLK,), jnp.int32),
            pltpu.VMEM((NW, 16), f32),
            pltpu.VMEM((S,), f32),
        ],
    )
    dpart = k2(w_arr, batch, mz)

    k3 = pl.kernel(
        _k3_body,
        out_type=jax.ShapeDtypeStruct((NW, S, D), f32),
        mesh=mesh,
        compiler_params=pltpu.CompilerParams(needs_layout_passes=False),
        scratch_types=[
            pltpu.VMEM((BLK, D), f32),
            pltpu.VMEM((BLK,), f32),
            pltpu.VMEM((BLK,), jnp.int32),
            pltpu.VMEM((NW, 16), f32),
            pltpu.VMEM((NW, S), f32),
            pltpu.VMEM((S,), f32),
            pltpu.VMEM((S, D), f32),
        ],
    )
    acc = k3(x, w_arr, batch, mz, dpart)

    pooled = pl.pallas_call(
        _k4_body,
        out_shape=jax.ShapeDtypeStruct((S, D), f32),
    )(acc)
    return pooled


# speedup vs baseline: 3.2534x; 2.1402x over previous
import functools

import jax
import jax.numpy as jnp
from jax import lax
from jax.experimental import pallas as pl
from jax.experimental.pallas import tpu as pltpu
from jax.experimental.pallas import tpu_sc as plsc

N = 100000
D = 128
S = 64
BLK = 160
NBLK = N // BLK
NW = 32
GRP = BLK // 16
NEG = -1e30


def _wid():
    return lax.axis_index("s") * 2 + lax.axis_index("c")


def _nblk(wid):
    return (NBLK - wid + NW - 1) // NW


def _mz_combine(mzv):
    iota = lax.iota(jnp.int32, 16)
    zero = jnp.zeros((16,), jnp.int32)
    one = zero + 1
    m1 = plsc.load_gather(mzv, [iota, zero])
    m2 = plsc.load_gather(mzv, [iota + 16, zero])
    z1 = plsc.load_gather(mzv, [iota, one])
    z2 = plsc.load_gather(mzv, [iota + 16, one])
    m = jnp.maximum(jnp.max(m1), jnp.max(m2))
    mv = jnp.full((16,), m, jnp.float32)
    z = jnp.sum(z1 * jnp.exp(m1 - mv) + z2 * jnp.exp(m2 - mv))
    zv = jnp.full((16,), z, jnp.float32)
    invzv = jnp.ones((16,), jnp.float32) / zv
    return mv, invzv


def _k1_body(x_hbm, wb_hbm, w_hbm, mz_hbm, xv, wblkv, wbv, tbuf, mzv):
    wid = _wid()
    pltpu.sync_copy(wb_hbm, wbv)
    iota = lax.iota(jnp.int32, 16)
    wq = [wbv[pl.ds(16 * q, 16)] for q in range(D // 16)]
    biasv = plsc.load_gather(wbv, [jnp.full((16,), D, jnp.int32)])
    m0 = jnp.full((16,), NEG, jnp.float32)
    z0 = jnp.zeros((16,), jnp.float32)

    def blk_body(i, carry):
        r0 = (wid + i * NW) * BLK
        pltpu.sync_copy(x_hbm.at[pl.ds(r0, BLK)], xv)

        def grp_body(g, carry2):
            m, z = carry2
            for l in range(16):
                r = 16 * g + l
                acc = xv[r, pl.ds(0, 16)] * wq[0]
                for q in range(1, D // 16):
                    acc = acc + xv[r, pl.ds(16 * q, 16)] * wq[q]
                tbuf[l, pl.ds(0, 16)] = acc
            wvec = biasv
            for c in range(16):
                wvec = wvec + plsc.load_gather(
                    tbuf, [iota, jnp.full((16,), c, jnp.int32)]
                )
            wblkv[pl.ds(16 * g, 16)] = wvec
            mn = jnp.maximum(m, wvec)
            z = z * jnp.exp(m - mn) + jnp.exp(wvec - mn)
            return (mn, z)

        carry = lax.fori_loop(0, GRP, grp_body, carry)
        pltpu.sync_copy(wblkv, w_hbm.at[pl.ds(r0, BLK)])
        return carry

    m, z = lax.fori_loop(0, _nblk(wid), blk_body, (m0, z0))
    mw = jnp.max(m)
    zw = jnp.sum(z * jnp.exp(m - mw))
    mzv[...] = jnp.where(iota == 0, mw, jnp.where(iota == 1, zw, 0.0))
    pltpu.sync_copy(mzv, mz_hbm.at[wid])


def _k2_body(w_hbm, b_hbm, mz_hbm, dp_hbm, wv, bv, mzv, denomv):
    wid = _wid()
    pltpu.sync_copy(mz_hbm, mzv)
    mv, invzv = _mz_combine(mzv)
    zero16 = jnp.zeros((16,), jnp.float32)
    for c in range(S // 16):
        denomv[pl.ds(16 * c, 16)] = zero16

    def blk_body(i, _):
        r0 = (wid + i * NW) * BLK
        pltpu.sync_copy(w_hbm.at[pl.ds(r0, BLK)], wv)
        pltpu.sync_copy(b_hbm.at[pl.ds(r0, BLK)], bv)
        for g in range(GRP):
            wvec = wv[pl.ds(16 * g, 16)]
            ev = jnp.exp(jnp.exp(wvec - mv) * invzv)
            plsc.addupdate_scatter(denomv, [bv[pl.ds(16 * g, 16)]], ev)
        return 0

    lax.fori_loop(0, _nblk(wid), blk_body, 0)
    pltpu.sync_copy(denomv, dp_hbm.at[wid])


def _k3_body(x_hbm, w_hbm, b_hbm, mz_hbm, dp_hbm, acc_hbm,
             xv, wv, bv, mzv, dpv, cinvv, accv):
    wid = _wid()
    pltpu.sync_copy(mz_hbm, mzv)
    mv, invzv = _mz_combine(mzv)
    pltpu.sync_copy(dp_hbm, dpv)
    one16 = jnp.ones((16,), jnp.float32)
    for c in range(S // 16):
        s = dpv[0, pl.ds(16 * c, 16)]
        for r in range(1, NW):
            s = s + dpv[r, pl.ds(16 * c, 16)]
        cinvv[pl.ds(16 * c, 16)] = one16 / s

    zero16 = jnp.zeros((16,), jnp.float32)

    def zero_body(r, _):
        for j in range(D // 16):
            accv[r, pl.ds(16 * j, 16)] = zero16
        return 0

    lax.fori_loop(0, S, zero_body, 0)

    def blk_body(i, _):
        r0 = (wid + i * NW) * BLK
        pltpu.sync_copy(x_hbm.at[pl.ds(r0, BLK)], xv)
        pltpu.sync_copy(w_hbm.at[pl.ds(r0, BLK)], wv)
        pltpu.sync_copy(b_hbm.at[pl.ds(r0, BLK)], bv)

        def grp_body(g, _):
            wvec = wv[pl.ds(16 * g, 16)]
            bvec = bv[pl.ds(16 * g, 16)]
            gv = jnp.exp(wvec - mv) * invzv
            nw = jnp.exp(gv) * plsc.load_gather(cinvv, [bvec])

            def uniform():
                accs = [jnp.zeros((16,), jnp.float32) for _ in range(D // 16)]
                for l in range(16):
                    r = 16 * g + l
                    sv = jnp.full((16,), nw[l], jnp.float32)
                    for j in range(D // 16):
                        accs[j] = accs[j] + xv[r, pl.ds(16 * j, 16)] * sv
                bi = bvec[0]
                for j in range(D // 16):
                    plsc.addupdate(accv.at[bi, pl.ds(16 * j, 16)], accs[j])

            def mixed():
                for l in range(16):
                    bi = bvec[l]
                    sv = jnp.full((16,), nw[l], jnp.float32)
                    r = 16 * g + l
                    for j in range(D // 16):
                        plsc.addupdate(accv.at[bi, pl.ds(16 * j, 16)],
                                       xv[r, pl.ds(16 * j, 16)] * sv)

            lax.cond(bvec[0] == bvec[15], uniform, mixed)
            return 0

        lax.fori_loop(0, GRP, grp_body, 0)
        return 0

    lax.fori_loop(0, _nblk(wid), blk_body, 0)
    pltpu.sync_copy(accv, acc_hbm.at[wid])


def _k4_body(a_ref, o_ref):
    o_ref[...] = jnp.sum(a_ref[...], axis=0)


@jax.jit
def kernel(x, batch, W, b):
    f32 = jnp.float32
    wb = jnp.concatenate(
        [W.reshape(D).astype(f32), jnp.broadcast_to(b.astype(f32), (32,))]
    )
    mesh = plsc.VectorSubcoreMesh(core_axis_name="c", subcore_axis_name="s")

    k1 = pl.kernel(
        _k1_body,
        out_type=(
            jax.ShapeDtypeStruct((N,), f32),
            jax.ShapeDtypeStruct((NW, 16), f32),
        ),
        mesh=mesh,
        compiler_params=pltpu.CompilerParams(needs_layout_passes=False),
        scratch_types=[
            pltpu.VMEM((BLK, D), f32),
            pltpu.VMEM((BLK,), f32),
            pltpu.VMEM((D + 32,), f32),
            pltpu.VMEM((16, 17), f32),
            pltpu.VMEM((16,), f32),
        ],
    )
    w_arr, mz = k1(x, wb)

    k2 = pl.kernel(
        _k2_body,
        out_type=jax.ShapeDtypeStruct((NW, S), f32),
        mesh=mesh,
        compiler_params=pltpu.CompilerParams(needs_layout_passes=False),
        scratch_types=[
            pltpu.VMEM((BLK,), f32),
            pltpu.VMEM((BLK,), jnp.int32),
            pltpu.VMEM((NW, 16), f32),
            pltpu.VMEM((S,), f32),
        ],
    )
    dpart = k2(w_arr, batch, mz)

    k3 = pl.kernel(
        _k3_body,
        out_type=jax.ShapeDtypeStruct((NW, S, D), f32),
        mesh=mesh,
        compiler_params=pltpu.CompilerParams(needs_layout_passes=False),
        scratch_types=[
            pltpu.VMEM((BLK, D), f32),
            pltpu.VMEM((BLK,), f32),
            pltpu.VMEM((BLK,), jnp.int32),
            pltpu.VMEM((NW, 16), f32),
            pltpu.VMEM((NW, S), f32),
            pltpu.VMEM((S,), f32),
            pltpu.VMEM((S, D), f32),
        ],
    )
    acc = k3(x, w_arr, batch, mz, dpart)

    pooled = pl.pallas_call(
        _k4_body,
        out_shape=jax.ShapeDtypeStruct((S, D), f32),
    )(acc)
    return pooled


# speedup vs baseline: 5.1668x; 1.5881x over previous
import jax
import jax.numpy as jnp
from jax import lax
from jax.experimental import pallas as pl
from jax.experimental.pallas import tpu as pltpu
from jax.experimental.pallas import tpu_sc as plsc

N = 100000
D = 128
S = 64
BLK = 160
NBLK = N // BLK
NW = 32
GRP = BLK // 16
CAP = (NBLK + NW - 1) // NW
WLEN = CAP * BLK
NEG = -1e30


def _wid():
    return lax.axis_index("s") * 2 + lax.axis_index("c")


def _nblk(wid):
    return (NBLK - wid + NW - 1) // NW


def _mz_combine(mzv):
    iota = lax.iota(jnp.int32, 16)
    zero = jnp.zeros((16,), jnp.int32)
    one = zero + 1
    m1 = plsc.load_gather(mzv, [iota, zero])
    m2 = plsc.load_gather(mzv, [iota + 16, zero])
    z1 = plsc.load_gather(mzv, [iota, one])
    z2 = plsc.load_gather(mzv, [iota + 16, one])
    m = jnp.maximum(jnp.max(m1), jnp.max(m2))
    mv = jnp.full((16,), m, jnp.float32)
    z = jnp.sum(z1 * jnp.exp(m1 - mv) + z2 * jnp.exp(m2 - mv))
    zv = jnp.full((16,), z, jnp.float32)
    invzv = jnp.ones((16,), jnp.float32) / zv
    return mv, invzv


def _k1_body(x_hbm, wb_hbm, w_hbm, mz_hbm,
             xv0, xv1, wall, wbv, tbuf, mzv, sem0, sem1):
    wid = _wid()
    nblk = _nblk(wid)
    pltpu.async_copy(x_hbm.at[pl.ds(wid * BLK, BLK)], xv0, sem0)
    pltpu.sync_copy(wb_hbm, wbv)
    iota = lax.iota(jnp.int32, 16)
    wq = [wbv[pl.ds(16 * q, 16)] for q in range(D // 16)]
    biasv = plsc.load_gather(wbv, [jnp.full((16,), D, jnp.int32)])
    m0 = jnp.full((16,), NEG, jnp.float32)
    z0 = jnp.zeros((16,), jnp.float32)

    def blk_body(i, carry):
        even = (i % 2) == 0
        nxt = i + 1

        @pl.when(jnp.logical_and(nxt < nblk, even))
        def _():
            pltpu.async_copy(
                x_hbm.at[pl.ds((wid + nxt * NW) * BLK, BLK)], xv1, sem1)

        @pl.when(jnp.logical_and(nxt < nblk, jnp.logical_not(even)))
        def _():
            pltpu.async_copy(
                x_hbm.at[pl.ds((wid + nxt * NW) * BLK, BLK)], xv0, sem0)

        def mk(xv, sem):
            def go():
                pltpu.make_async_copy(x_hbm.at[pl.ds(0, BLK)], xv, sem).wait()

                def grp_body(g, c2):
                    m, z = c2
                    for l in range(16):
                        r = 16 * g + l
                        acc = xv[r, pl.ds(0, 16)] * wq[0]
                        for q in range(1, D // 16):
                            acc = acc + xv[r, pl.ds(16 * q, 16)] * wq[q]
                        tbuf[l, pl.ds(0, 16)] = acc
                    wvec = biasv
                    for c in range(16):
                        wvec = wvec + plsc.load_gather(
                            tbuf, [iota, jnp.full((16,), c, jnp.int32)])
                    wall[pl.ds(i * BLK + 16 * g, 16)] = wvec
                    mn = jnp.maximum(m, wvec)
                    z2 = z * jnp.exp(m - mn) + jnp.exp(wvec - mn)
                    return (mn, z2)

                return lax.fori_loop(0, GRP, grp_body, carry)
            return go

        return lax.cond(even, mk(xv0, sem0), mk(xv1, sem1))

    m, z = lax.fori_loop(0, nblk, blk_body, (m0, z0))
    pltpu.sync_copy(wall, w_hbm.at[wid])
    mw = jnp.max(m)
    zw = jnp.sum(z * jnp.exp(m - jnp.full((16,), mw, jnp.float32)))
    mzv[...] = jnp.where(iota == 0, mw, jnp.where(iota == 1, zw, 0.0))
    pltpu.sync_copy(mzv, mz_hbm.at[wid])


def _k2_body(w_hbm, b_hbm, mz_hbm, dp_hbm, wbuf, bbuf, mzv, denomv):
    wid = _wid()
    nblk = _nblk(wid)
    pltpu.sync_copy(w_hbm.at[wid], wbuf)
    pltpu.sync_copy(b_hbm.at[wid], bbuf)
    pltpu.sync_copy(mz_hbm, mzv)
    mv, invzv = _mz_combine(mzv)
    zero16 = jnp.zeros((16,), jnp.float32)
    for c in range(S // 16):
        denomv[pl.ds(16 * c, 16)] = zero16

    def grp_body(g, _):
        wvec = wbuf[pl.ds(16 * g, 16)]
        ev = jnp.exp(jnp.exp(wvec - mv) * invzv)
        plsc.addupdate_scatter(denomv, [bbuf[pl.ds(16 * g, 16)]], ev)
        return 0

    lax.fori_loop(0, nblk * GRP, grp_body, 0)
    pltpu.sync_copy(denomv, dp_hbm.at[wid])


def _k3_body(x_hbm, w_hbm, b_hbm, mz_hbm, dp_hbm, acc_hbm,
             xv0, xv1, wbuf, bbuf, mzv, dpv, cinvv, accv, sem0, sem1):
    wid = _wid()
    nblk = _nblk(wid)
    pltpu.async_copy(x_hbm.at[pl.ds(wid * BLK, BLK)], xv0, sem0)
    pltpu.sync_copy(w_hbm.at[wid], wbuf)
    pltpu.sync_copy(b_hbm.at[wid], bbuf)
    pltpu.sync_copy(mz_hbm, mzv)
    mv, invzv = _mz_combine(mzv)
    pltpu.sync_copy(dp_hbm, dpv)
    one16 = jnp.ones((16,), jnp.float32)
    for c in range(S // 16):
        s = dpv[0, pl.ds(16 * c, 16)]
        for r in range(1, NW):
            s = s + dpv[r, pl.ds(16 * c, 16)]
        cinvv[pl.ds(16 * c, 16)] = one16 / s

    zero16 = jnp.zeros((16,), jnp.float32)

    def zero_body(r, _):
        for j in range(D // 16):
            accv[r, pl.ds(16 * j, 16)] = zero16
        return 0

    lax.fori_loop(0, S, zero_body, 0)

    def blk_body(i, _):
        even = (i % 2) == 0
        nxt = i + 1

        @pl.when(jnp.logical_and(nxt < nblk, even))
        def _():
            pltpu.async_copy(
                x_hbm.at[pl.ds((wid + nxt * NW) * BLK, BLK)], xv1, sem1)

        @pl.when(jnp.logical_and(nxt < nblk, jnp.logical_not(even)))
        def _():
            pltpu.async_copy(
                x_hbm.at[pl.ds((wid + nxt * NW) * BLK, BLK)], xv0, sem0)

        def mk(xv, sem):
            def go():
                pltpu.make_async_copy(x_hbm.at[pl.ds(0, BLK)], xv, sem).wait()

                def grp_body(g, _):
                    wvec = wbuf[pl.ds(i * BLK + 16 * g, 16)]
                    bvec = bbuf[pl.ds(i * BLK + 16 * g, 16)]
                    gv = jnp.exp(wvec - mv) * invzv
                    nw = jnp.exp(gv) * plsc.load_gather(cinvv, [bvec])

                    def uniform():
                        accs = [jnp.zeros((16,), jnp.float32)
                                for _ in range(D // 16)]
                        for l in range(16):
                            r = 16 * g + l
                            sv = jnp.full((16,), nw[l], jnp.float32)
                            for j in range(D // 16):
                                accs[j] = accs[j] + xv[r, pl.ds(16 * j, 16)] * sv
                        bi = bvec[0]
                        for j in range(D // 16):
                            plsc.addupdate(accv.at[bi, pl.ds(16 * j, 16)],
                                           accs[j])

                    def mixed():
                        for l in range(16):
                            bi = bvec[l]
                            sv = jnp.full((16,), nw[l], jnp.float32)
                            r = 16 * g + l
                            for j in range(D // 16):
                                plsc.addupdate(
                                    accv.at[bi, pl.ds(16 * j, 16)],
                                    xv[r, pl.ds(16 * j, 16)] * sv)

                    lax.cond(bvec[0] == bvec[15], uniform, mixed)
                    return 0

                return lax.fori_loop(0, GRP, grp_body, 0)
            return go

        return lax.cond(even, mk(xv0, sem0), mk(xv1, sem1))

    lax.fori_loop(0, nblk, blk_body, 0)
    pltpu.sync_copy(accv, acc_hbm.at[wid])


def _k4_body(a_ref, o_ref):
    o_ref[...] = jnp.sum(a_ref[...], axis=0)


@jax.jit
def kernel(x, batch, W, b):
    f32 = jnp.float32
    i32 = jnp.int32
    wb = jnp.concatenate(
        [W.reshape(D).astype(f32), jnp.broadcast_to(b.astype(f32), (32,))]
    )
    b2 = (
        jnp.pad(batch.astype(i32), (0, CAP * NW * BLK - N))
        .reshape(CAP, NW, BLK)
        .transpose(1, 0, 2)
        .reshape(NW, WLEN)
    )
    mesh = plsc.VectorSubcoreMesh(core_axis_name="c", subcore_axis_name="s")
    params = pltpu.CompilerParams(needs_layout_passes=False)

    k1 = pl.kernel(
        _k1_body,
        out_type=(
            jax.ShapeDtypeStruct((NW, WLEN), f32),
            jax.ShapeDtypeStruct((NW, 16), f32),
        ),
        mesh=mesh,
        compiler_params=params,
        scratch_types=[
            pltpu.VMEM((BLK, D), f32),
            pltpu.VMEM((BLK, D), f32),
            pltpu.VMEM((WLEN,), f32),
            pltpu.VMEM((D + 32,), f32),
            pltpu.VMEM((16, 17), f32),
            pltpu.VMEM((16,), f32),
            pltpu.SemaphoreType.DMA,
            pltpu.SemaphoreType.DMA,
        ],
    )
    w2, mz = k1(x, wb)

    k2 = pl.kernel(
        _k2_body,
        out_type=jax.ShapeDtypeStruct((NW, S), f32),
        mesh=mesh,
        compiler_params=params,
        scratch_types=[
            pltpu.VMEM((WLEN,), f32),
            pltpu.VMEM((WLEN,), i32),
            pltpu.VMEM((NW, 16), f32),
            pltpu.VMEM((S,), f32),
        ],
    )
    dpart = k2(w2, b2, mz)

    k3 = pl.kernel(
        _k3_body,
        out_type=jax.ShapeDtypeStruct((NW, S, D), f32),
        mesh=mesh,
        compiler_params=params,
        scratch_types=[
            pltpu.VMEM((BLK, D), f32),
            pltpu.VMEM((BLK, D), f32),
            pltpu.VMEM((WLEN,), f32),
            pltpu.VMEM((WLEN,), i32),
            pltpu.VMEM((NW, 16), f32),
            pltpu.VMEM((NW, S), f32),
            pltpu.VMEM((S,), f32),
            pltpu.VMEM((S, D), f32),
            pltpu.SemaphoreType.DMA,
            pltpu.SemaphoreType.DMA,
        ],
    )
    acc = k3(x, w2, b2, mz, dpart)

    pooled = pl.pallas_call(
        _k4_body,
        out_shape=jax.ShapeDtypeStruct((S, D), f32),
    )(acc)
    return pooled
